# Initial kernel scaffold; baseline (speedup 1.0000x reference)
import jax
import jax.numpy as jnp
from jax.experimental import pallas as pl


def kernel(inputs, embeddings_weight):
    raise NotImplementedError("write your pallas kernel here")


# speedup vs baseline: 10.9147x; 10.9147x over previous
import functools

import jax
import jax.numpy as jnp
from jax import lax
from jax.experimental import pallas as pl
from jax.experimental.pallas import tpu as pltpu
from jax.experimental.pallas import tpu_sc as plsc

N_TOK = 8192
K = 8192
D = 32
TOK_TILE = 1024
K_CHUNK = 2048
COMMIT = 0.25

_NC = 2
_NS = 16
_NW = _NC * _NS
_BPW = N_TOK // _NW
_CH = 128
_NCH = _BPW // _CH
_DP = 128


def _argmin_body(x_ref, e_ref, idx_ref):
    x = x_ref[...]
    x2 = jnp.sum(x * x, axis=1, keepdims=True)

    def body(c, carry):
        run_min, run_idx = carry
        ech = e_ref[pl.ds(c * K_CHUNK, K_CHUNK), :]
        e2 = jnp.sum(ech * ech, axis=1)
        mm = jax.lax.dot_general(
            x, ech, (((1,), (1,)), ((), ())),
            preferred_element_type=jnp.float32)
        d = (x2 + e2[None, :]) - 2.0 * mm
        lmin = jnp.min(d, axis=1, keepdims=True)
        iota = jax.lax.broadcasted_iota(jnp.int32, d.shape, 1) + c * K_CHUNK
        lidx = jnp.min(jnp.where(d == lmin, iota, jnp.int32(2 ** 30)),
                       axis=1, keepdims=True)
        upd = lmin < run_min
        return (jnp.where(upd, lmin, run_min), jnp.where(upd, lidx, run_idx))

    init = (jnp.full((TOK_TILE, 1), jnp.inf, jnp.float32),
            jnp.zeros((TOK_TILE, 1), jnp.int32))
    _, run_idx = jax.lax.fori_loop(0, K // K_CHUNK, body, init)
    idx_ref[0] = run_idx


def _argmin_indices(flat, emb):
    n_tiles = N_TOK // TOK_TILE
    idx3 = pl.pallas_call(
        _argmin_body,
        grid=(n_tiles,),
        in_specs=[
            pl.BlockSpec((TOK_TILE, D), lambda i: (i, 0)),
            pl.BlockSpec((K, D), lambda i: (0, 0)),
        ],
        out_specs=pl.BlockSpec((1, TOK_TILE, 1), lambda i: (i, 0, 0)),
        out_shape=jax.ShapeDtypeStruct((n_tiles, TOK_TILE, 1), jnp.int32),
    )(flat, emb)
    return idx3.reshape(N_TOK // _CH, _CH)


def _sc_body(e_hbm, idx_hbm, q_hbm, cnt_hbm, idx_v, rows_v, ones_v, zeros_v,
             cnt_sh, sem):
    cid = lax.axis_index("c")
    sid = lax.axis_index("s")
    wid = sid * _NC + cid
    base = wid * _BPW

    pltpu.sync_copy(idx_hbm.at[pl.ds(wid * _NCH, _NCH)], idx_v)

    for j in range(_NCH):
        pltpu.async_copy(e_hbm.at[idx_v.at[j]],
                         rows_v.at[pl.ds(j * _CH, _CH)], sem).wait()
    pltpu.sync_copy(rows_v, q_hbm.at[pl.ds(base, _BPW)])

    for i in range(_CH // 16):
        ones_v[pl.ds(i * 16, 16)] = jnp.full((16,), 1.0, jnp.float32)
    for i in range(512 // 16):
        zeros_v[pl.ds(i * 16, 16)] = jnp.zeros((16,), jnp.float32)
    pltpu.sync_copy(zeros_v, cnt_sh.at[pl.ds(sid * 512, 512)])
    plsc.subcore_barrier()
    for j in range(_NCH):
        pltpu.sync_copy(ones_v, cnt_sh.at[idx_v.at[j]], add=True)
    plsc.subcore_barrier()

    @pl.when(sid == 0)
    def _flush():
        pltpu.sync_copy(cnt_sh, cnt_hbm.at[cid])


@functools.partial(
    pl.kernel,
    out_type=[jax.ShapeDtypeStruct((N_TOK, _DP), jnp.float32),
              jax.ShapeDtypeStruct((_NC, K), jnp.float32)],
    mesh=plsc.VectorSubcoreMesh(core_axis_name="c", subcore_axis_name="s",
                                num_cores=_NC, num_subcores=_NS),
    scratch_types=[
        pltpu.VMEM((_NCH, _CH), jnp.int32),
        pltpu.VMEM((_BPW, _DP), jnp.float32),
        pltpu.VMEM((_CH,), jnp.float32),
        pltpu.VMEM((512,), jnp.float32),
        pltpu.VMEM_SHARED((K,), jnp.float32),
        pltpu.SemaphoreType.DMA,
    ],
)
def _sc_gather_hist(e_hbm, idx_hbm, q_hbm, cnt_hbm, idx_v, rows_v, ones_v,
                    zeros_v, cnt_sh, sem):
    _sc_body(e_hbm, idx_hbm, q_hbm, cnt_hbm, idx_v, rows_v, ones_v, zeros_v,
             cnt_sh, sem)


def _fin_body(x_ref, q_ref, c_ref, qst_ref, loss_ref, perp_ref):
    x = x_ref[...]
    q = q_ref[:, 0:D]
    diff = q - x
    qst_ref[...] = x + diff
    m = jnp.sum(diff * diff) * (1.0 / (N_TOK * D))
    loss_ref[...] = jnp.reshape(m + COMMIT * m, (1, 1))
    cnt = c_ref[0:1, :] + c_ref[1:2, :]
    p = cnt * (1.0 / N_TOK)
    ent = p * jnp.log(p + 1e-10)
    perp_ref[...] = jnp.reshape(jnp.exp(-jnp.sum(ent)), (1, 1))


def _finalize(flat, q, counts):
    return pl.pallas_call(
        _fin_body,
        out_shape=[
            jax.ShapeDtypeStruct((N_TOK, D), jnp.float32),
            jax.ShapeDtypeStruct((1, 1), jnp.float32),
            jax.ShapeDtypeStruct((1, 1), jnp.float32),
        ],
    )(flat, q, counts)


def kernel(inputs, embeddings_weight):
    input_shape = inputs.shape
    flat = inputs.reshape(-1, D)
    idx = _argmin_indices(flat, embeddings_weight)
    e_pad = jnp.pad(embeddings_weight, ((0, 0), (0, _DP - D)))
    quantized, counts = _sc_gather_hist(e_pad, idx)
    qst, loss, perp = _finalize(flat, quantized, counts)
    return (qst.reshape(input_shape), loss.reshape(()), perp.reshape(()))


# speedup vs baseline: 11.6857x; 1.0706x over previous
import functools

import jax
import jax.numpy as jnp
from jax import lax
from jax.experimental import pallas as pl
from jax.experimental.pallas import tpu as pltpu
from jax.experimental.pallas import tpu_sc as plsc

N_TOK = 8192
K = 8192
D = 32
TOK_TILE = 1024
K_CHUNK = 2048
COMMIT = 0.25

_NC = 2
_NS = 16
_NW = _NC * _NS
_BPW = N_TOK // _NW
_CH = 128
_NCH = _BPW // _CH
_DP = 128


def _argmin_body(x_ref, e_ref, idx_ref):
    x = x_ref[...]
    x2 = jnp.sum(x * x, axis=1, keepdims=True)

    def block_winner(c):
        ech = e_ref[pl.ds(c * K_CHUNK, K_CHUNK), :]
        e2 = jnp.sum(ech * ech, axis=1)
        mm = jax.lax.dot_general(
            x, ech, (((1,), (1,)), ((), ())),
            preferred_element_type=jnp.float32)
        d = (x2 + e2[None, :]) - 2.0 * mm
        lmin = jnp.min(d, axis=1, keepdims=True)
        iota = jax.lax.broadcasted_iota(jnp.int32, d.shape, 1) + c * K_CHUNK
        lidx = jnp.min(jnp.where(d == lmin, iota, jnp.int32(2 ** 30)),
                       axis=1, keepdims=True)
        return lmin, lidx

    v0, w0 = block_winner(0)
    v1, w1 = block_winner(1)
    v2, w2 = block_winner(2)
    v3, w3 = block_winner(3)
    t1 = v1 < v0
    sv1 = jnp.where(t1, v1, v0)
    sw1 = jnp.where(t1, w1, w0)
    t2 = v3 < v2
    sv2 = jnp.where(t2, v3, v2)
    sw2 = jnp.where(t2, w3, w2)
    sv1_bf = sv1.astype(jnp.bfloat16).astype(jnp.float32)
    take2 = sv2 < sv1_bf
    idx_ref[0] = jnp.where(take2, sw2, sw1)


def _argmin_indices(flat, emb):
    n_tiles = N_TOK // TOK_TILE
    idx3 = pl.pallas_call(
        _argmin_body,
        grid=(n_tiles,),
        in_specs=[
            pl.BlockSpec((TOK_TILE, D), lambda i: (i, 0)),
            pl.BlockSpec((K, D), lambda i: (0, 0)),
        ],
        out_specs=pl.BlockSpec((1, TOK_TILE, 1), lambda i: (i, 0, 0)),
        out_shape=jax.ShapeDtypeStruct((n_tiles, TOK_TILE, 1), jnp.int32),
    )(flat, emb)
    return idx3.reshape(N_TOK // _CH, _CH)


def _sc_body(e_hbm, idx_hbm, q_hbm, cnt_hbm, idx_v, rows_v, ones_v, zeros_v,
             cnt_sh, sem):
    cid = lax.axis_index("c")
    sid = lax.axis_index("s")
    wid = sid * _NC + cid
    base = wid * _BPW

    pltpu.sync_copy(idx_hbm.at[pl.ds(wid * _NCH, _NCH)], idx_v)

    for j in range(_NCH):
        pltpu.async_copy(e_hbm.at[idx_v.at[j]],
                         rows_v.at[pl.ds(j * _CH, _CH)], sem).wait()
    pltpu.sync_copy(rows_v, q_hbm.at[pl.ds(base, _BPW)])

    for i in range(_CH // 16):
        ones_v[pl.ds(i * 16, 16)] = jnp.full((16,), 1.0, jnp.float32)
    for i in range(512 // 16):
        zeros_v[pl.ds(i * 16, 16)] = jnp.zeros((16,), jnp.float32)
    pltpu.sync_copy(zeros_v, cnt_sh.at[pl.ds(sid * 512, 512)])
    plsc.subcore_barrier()
    for j in range(_NCH):
        pltpu.sync_copy(ones_v, cnt_sh.at[idx_v.at[j]], add=True)
    plsc.subcore_barrier()

    @pl.when(sid == 0)
    def _flush():
        pltpu.sync_copy(cnt_sh, cnt_hbm.at[cid])


@functools.partial(
    pl.kernel,
    out_type=[jax.ShapeDtypeStruct((N_TOK, _DP), jnp.float32),
              jax.ShapeDtypeStruct((_NC, K), jnp.float32)],
    mesh=plsc.VectorSubcoreMesh(core_axis_name="c", subcore_axis_name="s",
                                num_cores=_NC, num_subcores=_NS),
    scratch_types=[
        pltpu.VMEM((_NCH, _CH), jnp.int32),
        pltpu.VMEM((_BPW, _DP), jnp.float32),
        pltpu.VMEM((_CH,), jnp.float32),
        pltpu.VMEM((512,), jnp.float32),
        pltpu.VMEM_SHARED((K,), jnp.float32),
        pltpu.SemaphoreType.DMA,
    ],
)
def _sc_gather_hist(e_hbm, idx_hbm, q_hbm, cnt_hbm, idx_v, rows_v, ones_v,
                    zeros_v, cnt_sh, sem):
    _sc_body(e_hbm, idx_hbm, q_hbm, cnt_hbm, idx_v, rows_v, ones_v, zeros_v,
             cnt_sh, sem)


def _fin_body(x_ref, q_ref, c_ref, qst_ref, loss_ref, perp_ref):
    x = x_ref[...]
    q = q_ref[:, 0:D]
    diff = q - x
    qst_ref[...] = x + diff
    m = jnp.sum(diff * diff) * (1.0 / (N_TOK * D))
    loss_ref[...] = jnp.reshape(m + COMMIT * m, (1, 1))
    cnt = c_ref[0:1, :] + c_ref[1:2, :]
    p = cnt * (1.0 / N_TOK)
    ent = p * jnp.log(p + 1e-10)
    perp_ref[...] = jnp.reshape(jnp.exp(-jnp.sum(ent)), (1, 1))


def _finalize(flat, q, counts):
    return pl.pallas_call(
        _fin_body,
        out_shape=[
            jax.ShapeDtypeStruct((N_TOK, D), jnp.float32),
            jax.ShapeDtypeStruct((1, 1), jnp.float32),
            jax.ShapeDtypeStruct((1, 1), jnp.float32),
        ],
    )(flat, q, counts)


def kernel(inputs, embeddings_weight):
    input_shape = inputs.shape
    flat = inputs.reshape(-1, D)
    idx = _argmin_indices(flat, embeddings_weight)
    e_pad = jnp.pad(embeddings_weight, ((0, 0), (0, _DP - D)))
    quantized, counts = _sc_gather_hist(e_pad, idx)
    qst, loss, perp = _finalize(flat, quantized, counts)
    return (qst.reshape(input_shape), loss.reshape(()), perp.reshape(()))
